# baseline (speedup 1.0000x reference)
import jax
import jax.numpy as jnp
from jax.experimental import pallas as pl


def _pos_body(row_ref, col_ref, out_ref):
    _, n_dim, H, W = out_ref.shape
    e = n_dim // 2
    col_t = col_ref[:W, :].T
    row_t = row_ref[:H, :].T
    out_ref[0, :e, :, :] = jnp.broadcast_to(col_t[:, None, :], (e, H, W))
    out_ref[0, e:, :, :] = jnp.broadcast_to(row_t[:, :, None], (e, H, W))


def kernel(x, row_embed, col_embed):
    B = x.shape[0]
    H, W = x.shape[-2], x.shape[-1]
    e = row_embed.shape[1]
    n_dim = 2 * e
    return pl.pallas_call(
        _pos_body,
        grid=(B,),
        in_specs=[
            pl.BlockSpec(row_embed.shape, lambda b: (0, 0)),
            pl.BlockSpec(col_embed.shape, lambda b: (0, 0)),
        ],
        out_specs=pl.BlockSpec((1, n_dim, H, W), lambda b: (b, 0, 0, 0)),
        out_shape=jax.ShapeDtypeStruct((B, n_dim, H, W), row_embed.dtype),
    )(row_embed, col_embed)


# speedup vs baseline: 1.6950x; 1.6950x over previous
import functools

import jax
import jax.numpy as jnp
from jax.experimental import pallas as pl


def _pos_body(row_ref, col_ref, out_ref, *, H, W):
    _, n_dim, HW = out_ref.shape
    e = n_dim // 2
    col_t = col_ref[:W, :].T
    row_t = row_ref[:H, :].T
    top = jnp.broadcast_to(col_t[:, None, :], (e, H, W)).reshape(e, HW)
    bot = jnp.broadcast_to(row_t[:, :, None], (e, H, W)).reshape(e, HW)
    out_ref[0, :e, :] = top
    out_ref[0, e:, :] = bot


def kernel(x, row_embed, col_embed):
    B = x.shape[0]
    H, W = x.shape[-2], x.shape[-1]
    e = row_embed.shape[1]
    n_dim = 2 * e
    out = pl.pallas_call(
        functools.partial(_pos_body, H=H, W=W),
        grid=(B,),
        in_specs=[
            pl.BlockSpec(row_embed.shape, lambda b: (0, 0)),
            pl.BlockSpec(col_embed.shape, lambda b: (0, 0)),
        ],
        out_specs=pl.BlockSpec((1, n_dim, H * W), lambda b: (b, 0, 0)),
        out_shape=jax.ShapeDtypeStruct((B, n_dim, H * W), row_embed.dtype),
    )(row_embed, col_embed)
    return out.reshape(B, n_dim, H, W)


# speedup vs baseline: 2.6815x; 1.5820x over previous
import functools

import jax
import jax.numpy as jnp
from jax.experimental import pallas as pl
from jax.experimental.pallas import tpu as pltpu


def _pos_body(row_ref, col_ref, out_hbm, scratch, sem, *, H, W, B):
    n_dim, HW = scratch.shape
    e = n_dim // 2
    col_t = col_ref[:W, :].T
    row_t = row_ref[:H, :].T
    scratch[:e, :] = jnp.broadcast_to(col_t[:, None, :], (e, H, W)).reshape(e, HW)
    scratch[e:, :] = jnp.broadcast_to(row_t[:, :, None], (e, H, W)).reshape(e, HW)
    for b in range(B):
        pltpu.make_async_copy(scratch, out_hbm.at[b], sem).start()
    for b in range(B):
        pltpu.make_async_copy(scratch, out_hbm.at[b], sem).wait()


def kernel(x, row_embed, col_embed):
    B = x.shape[0]
    H, W = x.shape[-2], x.shape[-1]
    e = row_embed.shape[1]
    n_dim = 2 * e
    out = pl.pallas_call(
        functools.partial(_pos_body, H=H, W=W, B=B),
        in_specs=[
            pl.BlockSpec(memory_space=pltpu.MemorySpace.VMEM),
            pl.BlockSpec(memory_space=pltpu.MemorySpace.VMEM),
        ],
        out_specs=pl.BlockSpec(memory_space=pltpu.MemorySpace.HBM),
        out_shape=jax.ShapeDtypeStruct((B, n_dim, H * W), row_embed.dtype),
        scratch_shapes=[
            pltpu.VMEM((n_dim, H * W), row_embed.dtype),
            pltpu.SemaphoreType.DMA,
        ],
    )(row_embed, col_embed)
    return out.reshape(B, n_dim, H, W)
